# baseline (speedup 1.0000x reference)
import jax
import jax.numpy as jnp
from jax import lax
from jax.experimental import pallas as pl
from jax.experimental.pallas import tpu as pltpu
from jax.experimental.pallas import tpu_sc as plsc

B = 16384
NS = 5
D = 64
L = 16
NC = 2
NSUB = 16
NW = NC * NSUB
BPW = B // NW
CH = 128
NCHUNK = BPW // CH


def _body(tgt_hbm, ctx_hbm, win_hbm, wout_hbm, out_hbm,
          tidx_v, cidx_v, vin_v, vout_v, outb_v, sem):
    wid = lax.axis_index("s") * NC + lax.axis_index("c")
    base = wid * BPW
    for c in range(NCHUNK):
        off = base + c * CH
        pltpu.sync_copy(tgt_hbm.at[pl.ds(off, CH)], tidx_v)
        pltpu.sync_copy(ctx_hbm.at[pl.ds(off * NS, CH * NS)], cidx_v)
        pltpu.async_copy(win_hbm.at[tidx_v], vin_v, sem).wait()
        for m in range(NS):
            pltpu.async_copy(wout_hbm.at[cidx_v.at[pl.ds(m * CH, CH)]],
                             vout_v.at[pl.ds(m * CH, CH)], sem).wait()

        lane = lax.iota(jnp.int32, L)
        for g in range(CH // L):
            row = jnp.full((L,), g * L, jnp.int32) + lane
            rowj = [row * NS + j for j in range(NS)]
            zero = jnp.zeros((L,), jnp.float32)

            def dstep(d, accs):
                dsplat = jnp.full((L,), d, jnp.int32)
                vin_d = plsc.load_gather(vin_v, [row, dsplat])
                return tuple(
                    accs[j] + vin_d * plsc.load_gather(vout_v, [rowj[j], dsplat])
                    for j in range(NS))

            accs = lax.fori_loop(0, D, dstep, (zero,) * NS, unroll=4)
            for j in range(NS):
                plsc.store_scatter(outb_v, [rowj[j]], accs[j])
        pltpu.sync_copy(outb_v, out_hbm.at[pl.ds(off * NS, CH * NS)])


def kernel(target, context, W_in, W_out):
    tgt = target.reshape(B).astype(jnp.int32)
    ctx = context.reshape(B * NS).astype(jnp.int32)
    mesh = plsc.VectorSubcoreMesh(core_axis_name="c", subcore_axis_name="s")
    k = pl.kernel(
        _body,
        mesh=mesh,
        compiler_params=pltpu.CompilerParams(needs_layout_passes=False,
                                             use_tc_tiling_on_sc=False),
        out_type=jax.ShapeDtypeStruct((B * NS,), jnp.float32),
        scratch_types=[
            pltpu.VMEM((CH,), jnp.int32),
            pltpu.VMEM((CH * NS,), jnp.int32),
            pltpu.VMEM((CH, D), jnp.float32),
            pltpu.VMEM((CH * NS, D), jnp.float32),
            pltpu.VMEM((CH * NS,), jnp.float32),
            pltpu.SemaphoreType.DMA,
        ],
    )
    out = k(tgt, ctx, W_in, W_out)
    return out.reshape(B, NS)
